# baseline (speedup 1.0000x reference)
import functools

import jax
import jax.numpy as jnp
from jax import lax
from jax.experimental import pallas as pl
from jax.experimental.pallas import tpu as pltpu
from jax.experimental.pallas import tpu_sc as plsc

_NROWS = 16384
_NCOLS = 200
_L = 16


def _sc_energy_shift(species_flat, energies, se_pad):
    info = plsc.get_sparse_core_info()
    nw = info.num_cores * info.num_subcores
    rows_w = _NROWS // nw
    groups = rows_w // _L
    jsteps = _NCOLS // 4

    mesh = plsc.VectorSubcoreMesh(core_axis_name="c", subcore_axis_name="s")

    @functools.partial(
        pl.kernel,
        mesh=mesh,
        out_type=(
            jax.ShapeDtypeStruct((_NROWS * _NCOLS,), jnp.int32),
            jax.ShapeDtypeStruct((_NROWS,), jnp.float32),
        ),
        compiler_params=pltpu.CompilerParams(needs_layout_passes=False),
        scratch_types=[
            pltpu.VMEM((rows_w * _NCOLS,), jnp.int32),
            pltpu.VMEM((4096,), jnp.float32),
            pltpu.VMEM((_L,), jnp.float32),
            pltpu.VMEM((rows_w,), jnp.float32),
            pltpu.VMEM((rows_w,), jnp.float32),
            pltpu.SemaphoreType.DMA,
        ],
    )
    def k(species_hbm, energies_hbm, se_hbm, sp_out_hbm, en_out_hbm,
          sp_v, tb_v, se_v, en_v, out_v, sem):
        wid = lax.axis_index("s") * info.num_cores + lax.axis_index("c")
        rbase = wid * rows_w
        ebase = rbase * _NCOLS
        pltpu.sync_copy(se_hbm, se_v)
        pltpu.sync_copy(energies_hbm.at[pl.ds(rbase, rows_w)], en_v)
        pltpu.sync_copy(species_hbm.at[pl.ds(ebase, rows_w * _NCOLS)], sp_v)

        out_dma = pltpu.async_copy(
            sp_v, sp_out_hbm.at[pl.ds(ebase, rows_w * _NCOLS)], sem)

        lanes = lax.iota(jnp.int32, _L)

        def build(i, carry):
            e = i * _L + lanes
            a = lax.shift_right_logical(e, 9)
            b = lax.shift_right_logical(e, 6) & 7
            c = lax.shift_right_logical(e, 3) & 7
            d = e & 7
            t = (plsc.load_gather(se_v, [a]) + plsc.load_gather(se_v, [b])
                 + plsc.load_gather(se_v, [c]) + plsc.load_gather(se_v, [d]))
            tb_v[pl.ds(i * _L, _L)] = t
            return carry

        lax.fori_loop(0, 4096 // _L, build, 0)

        def group(g, carry):
            rowoff = (g * _L + lanes) * _NCOLS
            accs = [jnp.zeros((_L,), jnp.float32) for _ in range(4)]
            for j in range(jsteps):
                i0 = rowoff + (j * 4)
                s0 = plsc.load_gather(sp_v, [i0])
                s1 = plsc.load_gather(sp_v, [i0 + 1])
                s2 = plsc.load_gather(sp_v, [i0 + 2])
                s3 = plsc.load_gather(sp_v, [i0 + 3])
                idx = (lax.shift_left(s0, 9) | lax.shift_left(s1, 6)
                       | lax.shift_left(s2, 3) | s3)
                accs[j % 4] = accs[j % 4] + plsc.load_gather(tb_v, [idx])
            acc = (accs[0] + accs[1]) + (accs[2] + accs[3])
            out_v[pl.ds(g * _L, _L)] = acc + en_v[pl.ds(g * _L, _L)]
            return carry

        lax.fori_loop(0, groups, group, 0)
        pltpu.sync_copy(out_v, en_out_hbm.at[pl.ds(rbase, rows_w)])
        out_dma.wait()

    return k(species_flat, energies, se_pad)


def kernel(species, energies, self_energies):
    sp_flat = species.reshape(-1).astype(jnp.int32)
    se_pad = jnp.concatenate(
        [self_energies.astype(jnp.float32),
         jnp.zeros((_L - self_energies.shape[0],), jnp.float32)]
    )
    sp_out, new_energies = _sc_energy_shift(
        sp_flat, energies.astype(jnp.float32), se_pad)
    return (sp_out.reshape(species.shape).astype(species.dtype), new_energies)


# speedup vs baseline: 1.4257x; 1.4257x over previous
import functools

import jax
import jax.numpy as jnp
from jax import lax
from jax.experimental import pallas as pl
from jax.experimental.pallas import tpu as pltpu
from jax.experimental.pallas import tpu_sc as plsc

_NROWS = 16384
_NCOLS = 200
_L = 16
_CH = 128


def _sc_energy_shift(species, energies, se_pad):
    info = plsc.get_sparse_core_info()
    nw = info.num_cores * info.num_subcores
    rows_w = _NROWS // nw
    nchunks = rows_w // _CH
    cgroups = _CH // _L
    jsteps = _NCOLS // 4

    mesh = plsc.VectorSubcoreMesh(core_axis_name="c", subcore_axis_name="s")

    @functools.partial(
        pl.kernel,
        mesh=mesh,
        out_type=jax.ShapeDtypeStruct((_NROWS,), jnp.float32),
        compiler_params=pltpu.CompilerParams(needs_layout_passes=False),
        scratch_types=[
            pltpu.VMEM((_CH, _NCOLS), jnp.int32),
            pltpu.VMEM((_CH, _NCOLS), jnp.int32),
            pltpu.VMEM((4096,), jnp.float32),
            pltpu.VMEM((_L,), jnp.float32),
            pltpu.VMEM((rows_w,), jnp.float32),
            pltpu.VMEM((rows_w,), jnp.float32),
            pltpu.SemaphoreType.DMA,
            pltpu.SemaphoreType.DMA,
        ],
    )
    def k(species_hbm, energies_hbm, se_hbm, en_out_hbm,
          sp0, sp1, tb_v, se_v, en_v, out_v, sem0, sem1):
        wid = lax.axis_index("s") * info.num_cores + lax.axis_index("c")
        rbase = wid * rows_w
        bufs = (sp0, sp1)
        sems = (sem0, sem1)
        dmas = [None, None]

        pltpu.sync_copy(se_hbm, se_v)
        dmas[0] = pltpu.async_copy(
            species_hbm.at[pl.ds(rbase, _CH), :], sp0, sem0)
        pltpu.sync_copy(energies_hbm.at[pl.ds(rbase, rows_w)], en_v)

        lanes = lax.iota(jnp.int32, _L)

        def build(i, carry):
            e = i * _L + lanes
            a = lax.shift_right_logical(e, 9)
            b = lax.shift_right_logical(e, 6) & 7
            c = lax.shift_right_logical(e, 3) & 7
            d = e & 7
            t = (plsc.load_gather(se_v, [a]) + plsc.load_gather(se_v, [b])
                 + plsc.load_gather(se_v, [c]) + plsc.load_gather(se_v, [d]))
            tb_v[pl.ds(i * _L, _L)] = t
            return carry

        lax.fori_loop(0, 4096 // _L, build, 0)

        for ci in range(nchunks):
            buf = bufs[ci % 2]
            dmas[ci % 2].wait()
            if ci + 1 < nchunks:
                dmas[(ci + 1) % 2] = pltpu.async_copy(
                    species_hbm.at[pl.ds(rbase + (ci + 1) * _CH, _CH), :],
                    bufs[(ci + 1) % 2], sems[(ci + 1) % 2])

            def group(g, carry, buf=buf, ci=ci):
                r16 = g * _L + lanes
                accs = [jnp.zeros((_L,), jnp.float32) for _ in range(4)]

                def step(j10, accs):
                    accs = list(accs)
                    for u in range(5):
                        j = j10 * 5 + u
                        cvec = jnp.full((_L,), 0, jnp.int32) + (j * 4)
                        s0 = plsc.load_gather(buf, [r16, cvec])
                        s1 = plsc.load_gather(buf, [r16, cvec + 1])
                        s2 = plsc.load_gather(buf, [r16, cvec + 2])
                        s3 = plsc.load_gather(buf, [r16, cvec + 3])
                        idx = (lax.shift_left(s0, 9) | lax.shift_left(s1, 6)
                               | lax.shift_left(s2, 3) | s3)
                        accs[u % 4] = accs[u % 4] + plsc.load_gather(tb_v, [idx])
                    return tuple(accs)

                accs = lax.fori_loop(0, jsteps // 5, step, tuple(accs))
                acc = (accs[0] + accs[1]) + (accs[2] + accs[3])
                o = ci * _CH + g * _L
                out_v[pl.ds(o, _L)] = acc + en_v[pl.ds(o, _L)]
                return carry

            lax.fori_loop(0, cgroups, group, 0)

        pltpu.sync_copy(out_v, en_out_hbm.at[pl.ds(rbase, rows_w)])

    return k(species, energies, se_pad)


def kernel(species, energies, self_energies):
    se_pad = jnp.concatenate(
        [self_energies.astype(jnp.float32),
         jnp.zeros((_L - self_energies.shape[0],), jnp.float32)]
    )
    new_energies = _sc_energy_shift(
        species.astype(jnp.int32), energies.astype(jnp.float32), se_pad)
    return (species, new_energies)


# speedup vs baseline: 1.6429x; 1.1524x over previous
import functools

import jax
import jax.numpy as jnp
from jax import lax
from jax.experimental import pallas as pl
from jax.experimental.pallas import tpu as pltpu
from jax.experimental.pallas import tpu_sc as plsc

_NROWS = 16384
_NCOLS = 200
_L = 16
_CH = 128


def _sc_energy_shift(species, energies, se_pad):
    info = plsc.get_sparse_core_info()
    nw = info.num_cores * info.num_subcores
    rows_w = _NROWS // nw
    nchunks = rows_w // _CH
    cgroups = _CH // _L
    jsteps = _NCOLS // 4

    mesh = plsc.VectorSubcoreMesh(core_axis_name="c", subcore_axis_name="s")

    @functools.partial(
        pl.kernel,
        mesh=mesh,
        out_type=jax.ShapeDtypeStruct((_NROWS,), jnp.float32),
        compiler_params=pltpu.CompilerParams(
            needs_layout_passes=False, use_tc_tiling_on_sc=False),
        scratch_types=[
            pltpu.VMEM((_CH, _NCOLS), jnp.int32),
            pltpu.VMEM((_CH, _NCOLS), jnp.int32),
            pltpu.VMEM((4096,), jnp.float32),
            pltpu.VMEM((_L,), jnp.float32),
            pltpu.VMEM((rows_w,), jnp.float32),
            pltpu.VMEM((rows_w,), jnp.float32),
            pltpu.SemaphoreType.DMA,
            pltpu.SemaphoreType.DMA,
        ],
    )
    def k(species_hbm, energies_hbm, se_hbm, en_out_hbm,
          sp0, sp1, tb_v, se_v, en_v, out_v, sem0, sem1):
        wid = lax.axis_index("s") * info.num_cores + lax.axis_index("c")
        rbase = wid * rows_w
        bufs = (sp0, sp1)
        sems = (sem0, sem1)
        dmas = [None, None]

        pltpu.sync_copy(se_hbm, se_v)
        dmas[0] = pltpu.async_copy(
            species_hbm.at[pl.ds(rbase, _CH), :], sp0, sem0)
        pltpu.sync_copy(energies_hbm.at[pl.ds(rbase, rows_w)], en_v)

        lanes = lax.iota(jnp.int32, _L)

        def build(i, carry):
            e = i * _L + lanes
            a = lax.shift_right_logical(e, 9)
            b = lax.shift_right_logical(e, 6) & 7
            c = lax.shift_right_logical(e, 3) & 7
            d = e & 7
            t = (plsc.load_gather(se_v, [a]) + plsc.load_gather(se_v, [b])
                 + plsc.load_gather(se_v, [c]) + plsc.load_gather(se_v, [d]))
            tb_v[pl.ds(i * _L, _L)] = t
            return carry

        lax.fori_loop(0, 4096 // _L, build, 0)

        for ci in range(nchunks):
            buf = bufs[ci % 2]
            dmas[ci % 2].wait()
            if ci + 1 < nchunks:
                dmas[(ci + 1) % 2] = pltpu.async_copy(
                    species_hbm.at[pl.ds(rbase + (ci + 1) * _CH, _CH), :],
                    bufs[(ci + 1) % 2], sems[(ci + 1) % 2])

            def group(g, carry, buf=buf, ci=ci):
                r16 = g * _L + lanes
                accs = [jnp.zeros((_L,), jnp.float32) for _ in range(4)]

                def step(j10, accs):
                    accs = list(accs)
                    cbase = j10 * 40
                    for u in range(10):
                        cvec = jnp.zeros((_L,), jnp.int32) + (cbase + u * 4)
                        s0 = plsc.load_gather(buf, [r16, cvec])
                        s1 = plsc.load_gather(buf, [r16, cvec + 1])
                        s2 = plsc.load_gather(buf, [r16, cvec + 2])
                        s3 = plsc.load_gather(buf, [r16, cvec + 3])
                        idx = (lax.shift_left(s0, 9) | lax.shift_left(s1, 6)
                               | lax.shift_left(s2, 3) | s3)
                        accs[u % 4] = accs[u % 4] + plsc.load_gather(tb_v, [idx])
                    return tuple(accs)

                accs = lax.fori_loop(0, jsteps // 10, step, tuple(accs))
                acc = (accs[0] + accs[1]) + (accs[2] + accs[3])
                o = ci * _CH + g * _L
                out_v[pl.ds(o, _L)] = acc + en_v[pl.ds(o, _L)]
                return carry

            lax.fori_loop(0, cgroups, group, 0)

        pltpu.sync_copy(out_v, en_out_hbm.at[pl.ds(rbase, rows_w)])

    return k(species, energies, se_pad)


def kernel(species, energies, self_energies):
    se_pad = jnp.concatenate(
        [self_energies.astype(jnp.float32),
         jnp.zeros((_L - self_energies.shape[0],), jnp.float32)]
    )
    new_energies = _sc_energy_shift(
        species.astype(jnp.int32), energies.astype(jnp.float32), se_pad)
    return (species, new_energies)
